# baseline (speedup 1.0000x reference)
import functools

import jax
import jax.numpy as jnp
from jax import lax
from jax.experimental import pallas as pl
from jax.experimental.pallas import tpu as pltpu
from jax.experimental.pallas import tpu_sc as plsc

EMBEDDING_DIM = 64
NUM_WORKERS = 32
CHUNK = 128


def _make_kernel(batch):
    assert batch % (NUM_WORKERS * CHUNK) == 0
    b_per_w = batch // NUM_WORKERS
    n_chunks = b_per_w // CHUNK
    mesh = plsc.VectorSubcoreMesh(core_axis_name="c", subcore_axis_name="s")

    @functools.partial(
        pl.kernel,
        mesh=mesh,
        out_type=jax.ShapeDtypeStruct((batch, EMBEDDING_DIM), jnp.float32),
        compiler_params=pltpu.CompilerParams(use_tc_tiling_on_sc=False),
        scratch_types=[
            pltpu.VMEM((CHUNK,), jnp.int32),
            pltpu.VMEM((CHUNK, EMBEDDING_DIM), jnp.float32),
            pltpu.SemaphoreType.DMA,
        ],
    )
    def k(idx_hbm, table_hbm, out_hbm, idx_v, rows_v, sem):
        wid = lax.axis_index("s") * 2 + lax.axis_index("c")
        base = wid * b_per_w

        def body(g, carry):
            off = base + g * CHUNK
            pltpu.sync_copy(idx_hbm.at[pl.ds(off, CHUNK)], idx_v)
            pltpu.async_copy(table_hbm.at[idx_v], rows_v, sem).wait()
            pltpu.sync_copy(rows_v, out_hbm.at[pl.ds(off, CHUNK)])
            return carry

        lax.fori_loop(0, n_chunks, body, 0)

    return k


def kernel(token_ids, weights):
    idx = token_ids.astype(jnp.int32).reshape(-1)
    out = _make_kernel(idx.shape[0])(idx, weights)
    return out.reshape(token_ids.shape + (EMBEDDING_DIM,))


# speedup vs baseline: 1.1292x; 1.1292x over previous
import functools

import jax
import jax.numpy as jnp
from jax import lax
from jax.experimental import pallas as pl
from jax.experimental.pallas import tpu as pltpu
from jax.experimental.pallas import tpu_sc as plsc

EMBEDDING_DIM = 64
NUM_WORKERS = 32
CHUNK = 256
NBUF = 4


def _make_kernel(batch):
    assert batch % (NUM_WORKERS * CHUNK * NBUF) == 0
    b_per_w = batch // NUM_WORKERS
    n_chunks = b_per_w // CHUNK
    n_outer = n_chunks // NBUF
    mesh = plsc.VectorSubcoreMesh(core_axis_name="c", subcore_axis_name="s")

    @functools.partial(
        pl.kernel,
        mesh=mesh,
        out_type=jax.ShapeDtypeStruct((batch, EMBEDDING_DIM), jnp.float32),
        compiler_params=pltpu.CompilerParams(use_tc_tiling_on_sc=False),
        scratch_types=[
            pltpu.VMEM((b_per_w,), jnp.int32),
            pltpu.VMEM((NBUF, CHUNK, EMBEDDING_DIM), jnp.float32),
            pltpu.SemaphoreType.DMA((NBUF,)),
            pltpu.SemaphoreType.DMA((NBUF,)),
        ],
    )
    def k(idx_hbm, table_hbm, out_hbm, idx_all, rows, gsem, osem):
        wid = lax.axis_index("s") * 2 + lax.axis_index("c")
        base = wid * b_per_w
        pltpu.sync_copy(idx_hbm.at[pl.ds(base, b_per_w)], idx_all)

        def gather(g, b):
            return pltpu.make_async_copy(
                table_hbm.at[idx_all.at[pl.ds(g * CHUNK, CHUNK)]],
                rows.at[b],
                gsem.at[b],
            )

        def out_copy(g, b):
            return pltpu.make_async_copy(
                rows.at[b],
                out_hbm.at[pl.ds(base + g * CHUNK, CHUNK)],
                osem.at[b],
            )

        for b in range(NBUF):
            gather(b, b).start()

        def outer(gg, carry):
            for b in range(NBUF):
                g = gg * NBUF + b
                gather(g, b).wait()
                out_copy(g, b).start()
                out_copy(g, b).wait()

                @pl.when(gg < n_outer - 1)
                def _():
                    gather(g + NBUF, b).start()

            return carry

        lax.fori_loop(0, n_outer, outer, 0)

    return k


def kernel(token_ids, weights):
    idx = token_ids.astype(jnp.int32).reshape(-1)
    out = _make_kernel(idx.shape[0])(idx, weights)
    return out.reshape(token_ids.shape + (EMBEDDING_DIM,))
